# baseline (speedup 1.0000x reference)
import functools

import jax
import jax.numpy as jnp
from jax import lax
from jax.experimental import pallas as pl
from jax.experimental.pallas import tpu as pltpu
from jax.experimental.pallas import tpu_sc as plsc

EMBED_DIM = 32
BATCH = 16384

NUM_WORKERS = 32
ROWS_PER_W = BATCH // NUM_WORKERS
CHUNK = 128
NCHUNK = ROWS_PER_W // CHUNK


def _sc_gather_body(user_hbm, item_hbm, users_hbm, pos_hbm, neg_hbm,
                    uout_hbm, pout_hbm, nout_hbm,
                    uidx, pidx, nidx, urows, prows, nrows, sem):
    wid = lax.axis_index("s") * 2 + lax.axis_index("c")
    base = wid * ROWS_PER_W

    pltpu.sync_copy(users_hbm.at[wid], uidx)
    pltpu.sync_copy(pos_hbm.at[wid], pidx)
    pltpu.sync_copy(neg_hbm.at[wid], nidx)

    copies = []
    for j in range(NCHUNK):
        dst = pl.ds(j * CHUNK, CHUNK)
        copies.append(pltpu.async_copy(user_hbm.at[uidx.at[j]], urows.at[dst], sem))
        copies.append(pltpu.async_copy(item_hbm.at[pidx.at[j]], prows.at[dst], sem))
        copies.append(pltpu.async_copy(item_hbm.at[nidx.at[j]], nrows.at[dst], sem))
    for c in copies:
        c.wait()

    out_slc = pl.ds(base, ROWS_PER_W)
    pltpu.sync_copy(urows, uout_hbm.at[out_slc])
    pltpu.sync_copy(prows, pout_hbm.at[out_slc])
    pltpu.sync_copy(nrows, nout_hbm.at[out_slc])


_rows_t = jax.ShapeDtypeStruct((BATCH, EMBED_DIM), jnp.float32)

_sc_gather = functools.partial(
    pl.kernel,
    mesh=plsc.VectorSubcoreMesh(core_axis_name="c", subcore_axis_name="s"),
    out_type=(_rows_t, _rows_t, _rows_t),
    compiler_params=pltpu.CompilerParams(use_tc_tiling_on_sc=False),
    scratch_types=[
        pltpu.VMEM((NCHUNK, CHUNK), jnp.int32),
        pltpu.VMEM((NCHUNK, CHUNK), jnp.int32),
        pltpu.VMEM((NCHUNK, CHUNK), jnp.int32),
        pltpu.VMEM((ROWS_PER_W, EMBED_DIM), jnp.float32),
        pltpu.VMEM((ROWS_PER_W, EMBED_DIM), jnp.float32),
        pltpu.VMEM((ROWS_PER_W, EMBED_DIM), jnp.float32),
        pltpu.SemaphoreType.DMA,
    ],
)(_sc_gather_body)


def _loss_body(u_ref, p_ref, n_ref, o_ref):
    u = u_ref[...]
    p = p_ref[...]
    n = n_ref[...]
    d = jnp.sum(u * (n - p), axis=1)
    sp = jnp.maximum(d, 0.0) + jnp.log1p(jnp.exp(-jnp.abs(d)))
    o_ref[...] = (jnp.sum(sp) * (1.0 / BATCH)).reshape(1, 1)


def kernel(embedding_user, embedding_item, users, pos, neg):
    users = users.astype(jnp.int32).reshape(NUM_WORKERS, NCHUNK, CHUNK)
    pos = pos.astype(jnp.int32).reshape(NUM_WORKERS, NCHUNK, CHUNK)
    neg = neg.astype(jnp.int32).reshape(NUM_WORKERS, NCHUNK, CHUNK)
    urows, prows, nrows = _sc_gather(embedding_user, embedding_item,
                                     users, pos, neg)
    loss = pl.pallas_call(
        _loss_body,
        out_shape=jax.ShapeDtypeStruct((1, 1), jnp.float32),
    )(urows, prows, nrows)
    return loss[0, 0]


# speedup vs baseline: 1.6688x; 1.6688x over previous
import functools

import jax
import jax.numpy as jnp
from jax import lax
from jax.experimental import pallas as pl
from jax.experimental.pallas import tpu as pltpu
from jax.experimental.pallas import tpu_sc as plsc

N_ROWS = 1000000
EMBED_DIM = 32
BATCH = 16384

NUM_WORKERS = 32
ROWS_PER_W = BATCH // NUM_WORKERS
CHUNK = 256
NCHUNK = ROWS_PER_W // CHUNK
QROWS = N_ROWS // 4

K0_COLS = 8192
K0_GRID = -(-N_ROWS // K0_COLS)
K0_OUT_ROWS = K0_COLS * EMBED_DIM // 128


def _repack_body(t_ref, o_ref):
    x = t_ref[...]
    pieces = [
        jnp.transpose(x[:, m * K0_OUT_ROWS:(m + 1) * K0_OUT_ROWS])
        for m in range(4)
    ]
    o_ref[...] = jnp.concatenate(pieces, axis=1)


def _repack(t):
    return pl.pallas_call(
        _repack_body,
        grid=(K0_GRID,),
        in_specs=[pl.BlockSpec((EMBED_DIM, K0_COLS), lambda i: (0, i))],
        out_specs=pl.BlockSpec((K0_OUT_ROWS, 128), lambda i: (i, 0)),
        out_shape=jax.ShapeDtypeStruct((K0_GRID * K0_OUT_ROWS, 128),
                                       jnp.float32),
    )(t)


def _sc_diff_body(qu_hbm, qi_hbm, users_hbm, pos_hbm, neg_hbm, out_hbm,
                  idx_u, idx_p, idx_n, q_u, q_p, q_n, off_u, off_p, off_n,
                  buf_u, buf_p, buf_n, diff, sem):
    wid = lax.axis_index("s") * 2 + lax.axis_index("c")
    base = wid * ROWS_PER_W

    tabs = ((idx_u, q_u, off_u, qu_hbm, buf_u),
            (idx_p, q_p, off_p, qi_hbm, buf_p),
            (idx_n, q_n, off_n, qi_hbm, buf_n))

    for src, (idx, _, _, _, _) in zip((users_hbm, pos_hbm, neg_hbm), tabs):
        pltpu.sync_copy(src.at[pl.ds(base, ROWS_PER_W)], idx)

    def prep_body(k, carry):
        for idx, q, off, _src, _buf in tabs:
            iv = idx[pl.ds(k * 16, 16)]
            q[pl.ds(k * 16, 16)] = ((iv >> 13) << 11) + (iv & 2047)
            off[pl.ds(k * 16, 16)] = ((iv >> 11) & 3) * 32
        return carry
    lax.fori_loop(0, ROWS_PER_W // 16, prep_body, 0)

    for h in range(NCHUNK):
        for _, q, _, src, buf in tabs:
            for c in range(CHUNK // 128):
                pltpu.async_copy(
                    src.at[q.at[pl.ds(h * CHUNK + c * 128, 128)]],
                    buf.at[pl.ds(c * 128, 128)], sem)
        for _, _, _, _, buf in tabs:
            pltpu.make_async_copy(qu_hbm.at[pl.ds(0, CHUNK)], buf, sem).wait()

        def dot_body(g, _):
            rows = g * 16 + lax.iota(jnp.int32, 16)
            pos0 = h * CHUNK + g * 16
            ou = off_u[pl.ds(pos0, 16)]
            op = off_p[pl.ds(pos0, 16)]
            on = off_n[pl.ds(pos0, 16)]
            acc = jnp.zeros((16,), jnp.float32)
            for d in range(EMBED_DIM):
                uu = plsc.load_gather(buf_u, [rows, ou + d])
                pp = plsc.load_gather(buf_p, [rows, op + d])
                nn = plsc.load_gather(buf_n, [rows, on + d])
                acc = acc + uu * (nn - pp)
            diff[pl.ds(pos0, 16)] = acc
            return _
        lax.fori_loop(0, CHUNK // 16, dot_body, 0)

    pltpu.sync_copy(diff, out_hbm.at[pl.ds(base, ROWS_PER_W)])


_sc_diff = functools.partial(
    pl.kernel,
    mesh=plsc.VectorSubcoreMesh(core_axis_name="c", subcore_axis_name="s"),
    out_type=jax.ShapeDtypeStruct((BATCH,), jnp.float32),
    compiler_params=pltpu.CompilerParams(needs_layout_passes=False),
    scratch_types=[
        pltpu.VMEM((ROWS_PER_W,), jnp.int32),
        pltpu.VMEM((ROWS_PER_W,), jnp.int32),
        pltpu.VMEM((ROWS_PER_W,), jnp.int32),
        pltpu.VMEM((ROWS_PER_W,), jnp.int32),
        pltpu.VMEM((ROWS_PER_W,), jnp.int32),
        pltpu.VMEM((ROWS_PER_W,), jnp.int32),
        pltpu.VMEM((ROWS_PER_W,), jnp.int32),
        pltpu.VMEM((ROWS_PER_W,), jnp.int32),
        pltpu.VMEM((ROWS_PER_W,), jnp.int32),
        pltpu.VMEM((CHUNK, 128), jnp.float32),
        pltpu.VMEM((CHUNK, 128), jnp.float32),
        pltpu.VMEM((CHUNK, 128), jnp.float32),
        pltpu.VMEM((ROWS_PER_W,), jnp.float32),
        pltpu.SemaphoreType.DMA,
    ],
)(_sc_diff_body)


def _loss_body(d_ref, o_ref):
    d = d_ref[...]
    sp = jnp.maximum(d, 0.0) + jnp.log1p(jnp.exp(-jnp.abs(d)))
    o_ref[...] = (jnp.sum(sp) * (1.0 / BATCH)).reshape(1, 1)


def kernel(embedding_user, embedding_item, users, pos, neg):
    qu = _repack(embedding_user.T)
    qi = _repack(embedding_item.T)
    users = users.astype(jnp.int32)
    pos = pos.astype(jnp.int32)
    neg = neg.astype(jnp.int32)
    diffs = _sc_diff(qu, qi, users, pos, neg)
    loss = pl.pallas_call(
        _loss_body,
        out_shape=jax.ShapeDtypeStruct((1, 1), jnp.float32),
    )(diffs.reshape(128, 128))
    return loss[0, 0]
